# baseline (speedup 1.0000x reference)
import jax
import jax.numpy as jnp
from jax import lax
from jax.experimental import pallas as pl
from jax.experimental.pallas import tpu as pltpu
from jax.experimental.pallas import tpu_sc as plsc

_F = 100000
_B = 16384
_W = 128
_D_GO, _D_BP, _D_TR, _D_EX, _D_BE = 3, 63, 3, 10, 16
_C_GO, _C_TR, _C_EX, _C_BP = 0, 3, 6, 16
_R_GO, _R_BP, _R_TR, _R_EX = 8, 64, 8, 16

_NS = 16
_NW = 32
_BPW = _B // _NW
_CH = 128
_GR = 256
_PW = 128
_NP = _F // _PW
_TW = _F - _NP * _PW
_PPT = -(-(_NP + 1) // _NS)

_TABLES = ((_C_GO, _D_GO), (_C_TR, _D_TR), (_C_EX, _D_EX), (_C_BP, _D_BP))

_mesh = plsc.VectorSubcoreMesh(core_axis_name="c", subcore_axis_name="s")


def _body(ids_hbm, be_hbm, go_hbm, tr_hbm, ex_hbm, bp_hbm, tail_hbm,
          be_t, go_t, bp_t, tr_t, ex_t,
          scr, idx_v, be_row, go_pan, tr_pan, ex_pan, bp_pan, rowbuf, rows_v,
          be_p, go_p, bp_p, tr_p, ex_p, sem, wsem):
    cid = lax.axis_index("c")
    tid = lax.axis_index("s")
    wid = tid * 2 + cid
    base = wid * _BPW
    iota = lax.iota(jnp.int32, 16)

    scr_ref = scr.at[cid]
    pans = ((go_pan, _C_GO, _D_GO), (tr_pan, _C_TR, _D_TR),
            (ex_pan, _C_EX, _D_EX), (bp_pan, _C_BP, _D_BP))

    def _transpose(nck):
        for pan, c0, d in pans:
            def _col(j, _, pan=pan, c0=c0):
                rowsj = jnp.full((16,), j, jnp.int32)
                colj = jnp.full((16,), c0 + j, jnp.int32)

                def _chunk(k, _):
                    lanes = k * 16 + iota
                    v = plsc.load_gather(pan, [rowsj, lanes])
                    plsc.store_scatter(rowbuf, [lanes, colj], v)
                    return 0

                lax.fori_loop(0, nck, _chunk, 0)
                return 0

            lax.fori_loop(0, d, _col, 0)

    def _panel(i, _):
        p = tid + i * _NS
        po = p * _PW

        @pl.when(p < _NP)
        def _do():
            pltpu.sync_copy(go_hbm.at[:, pl.ds(po, _PW)], go_pan)
            pltpu.sync_copy(tr_hbm.at[:, pl.ds(po, _PW)], tr_pan)
            pltpu.sync_copy(ex_hbm.at[:, pl.ds(po, _PW)], ex_pan)
            pltpu.sync_copy(bp_hbm.at[:, pl.ds(po, _PW)], bp_pan)
            _transpose(_PW // 16)
            pltpu.sync_copy(rowbuf, scr_ref.at[pl.ds(po, _PW)])

        @pl.when(p == _NP)
        def _tail():
            pltpu.sync_copy(tail_hbm, rowbuf.at[pl.ds(0, _TW)])
            pltpu.sync_copy(rowbuf.at[pl.ds(0, _TW)],
                            scr_ref.at[pl.ds(_NP * _PW, _TW)])
        return 0

    lax.fori_loop(0, _PPT, _panel, 0)
    plsc.subcore_barrier()

    pltpu.sync_copy(ids_hbm.at[pl.ds(base, _BPW)], idx_v)

    pltpu.sync_copy(be_hbm, be_row)
    vec = be_row[0, :]

    def _berow(d, _):
        val = jnp.sum(jnp.where(iota == d, vec, 0.0))
        bvec = jnp.full((16,), val, jnp.float32)

        def _fill(i, _):
            be_p[d, pl.ds(i * 16, 16)] = bvec
            return 0

        lax.fori_loop(0, _BPW // 16, _fill, 0)
        return 0

    lax.fori_loop(0, _D_BE, _berow, 0)
    writes = [pltpu.async_copy(be_p, be_t.at[:, pl.ds(base, _BPW)], wsem)]

    def _round(r, _):
        g1 = pltpu.async_copy(
            scr_ref.at[idx_v.at[pl.ds(r * _GR, _CH)]],
            rows_v.at[pl.ds(0, _CH)], sem)
        g2 = pltpu.async_copy(
            scr_ref.at[idx_v.at[pl.ds(r * _GR + _CH, _CH)]],
            rows_v.at[pl.ds(_CH, _CH)], sem)
        g1.wait()
        g2.wait()
        for dst, (c0, d) in zip((go_p, tr_p, ex_p, bp_p), _TABLES):
            def _col(j, _, dst=dst, c0=c0):
                colj = jnp.full((16,), c0 + j, jnp.int32)

                def _tile(i, _):
                    rows = i * 16 + iota
                    dst[j, pl.ds(r * _GR + i * 16, 16)] = plsc.load_gather(
                        rows_v, [rows, colj])
                    return 0

                lax.fori_loop(0, _GR // 16, _tile, 0)
                return 0

            lax.fori_loop(0, d, _col, 0)
        return 0

    lax.fori_loop(0, _BPW // _GR, _round, 0)

    for dst, out in ((go_p, go_t), (tr_p, tr_t), (ex_p, ex_t), (bp_p, bp_t)):
        writes.append(pltpu.async_copy(dst, out.at[:, pl.ds(base, _BPW)],
                                       wsem))
    for w in writes:
        w.wait()


def _build(interpret=False):
    return pl.kernel(
        _body,
        mesh=_mesh,
        compiler_params=pltpu.CompilerParams(use_tc_tiling_on_sc=True,
                                             needs_layout_passes=False),
        out_type=[
            jax.ShapeDtypeStruct((_D_BE, _B), jnp.float32),
            jax.ShapeDtypeStruct((_R_GO, _B), jnp.float32),
            jax.ShapeDtypeStruct((_R_BP, _B), jnp.float32),
            jax.ShapeDtypeStruct((_R_TR, _B), jnp.float32),
            jax.ShapeDtypeStruct((_R_EX, _B), jnp.float32),
        ],
        scratch_types=[
            pltpu.HBM((2, _F, _W), jnp.float32),
            pltpu.VMEM((_BPW,), jnp.int32),
            pltpu.VMEM((1, _D_BE), jnp.float32),
            pltpu.VMEM((_D_GO, _PW), jnp.float32),
            pltpu.VMEM((_D_TR, _PW), jnp.float32),
            pltpu.VMEM((_D_EX, _PW), jnp.float32),
            pltpu.VMEM((_D_BP, _PW), jnp.float32),
            pltpu.VMEM((_PW, _W), jnp.float32),
            pltpu.VMEM((_GR, _W), jnp.float32),
            pltpu.VMEM((_D_BE, _BPW), jnp.float32),
            pltpu.VMEM((_R_GO, _BPW), jnp.float32),
            pltpu.VMEM((_R_BP, _BPW), jnp.float32),
            pltpu.VMEM((_R_TR, _BPW), jnp.float32),
            pltpu.VMEM((_R_EX, _BPW), jnp.float32),
            pltpu.SemaphoreType.DMA,
            pltpu.SemaphoreType.DMA,
        ],
        interpret=interpret,
    )


_gather_all = _build()


def kernel(frame_ids, betas_w, global_orient_w, body_pose_w, transl_w,
           expression_w):
    ids = frame_ids.astype(jnp.int32)
    tail = jnp.pad(
        jnp.concatenate([global_orient_w[-_TW:], transl_w[-_TW:],
                         expression_w[-_TW:], body_pose_w[-_TW:]], axis=1),
        ((0, 0), (0, _W - _C_BP - _D_BP)))
    be_t, go_t, bp_t, tr_t, ex_t = _gather_all(
        ids, betas_w, global_orient_w.T, transl_w.T, expression_w.T,
        body_pose_w.T, tail)
    return (be_t.T, go_t[:_D_GO].T, bp_t[:_D_BP].T, tr_t[:_D_TR].T,
            ex_t[:_D_EX].T)


# speedup vs baseline: 1.9474x; 1.9474x over previous
import jax
import jax.numpy as jnp
from jax import lax
from jax.experimental import pallas as pl
from jax.experimental.pallas import tpu as pltpu
from jax.experimental.pallas import tpu_sc as plsc

_F = 100000
_B = 16384
_W = 128
_D_GO, _D_BP, _D_TR, _D_EX, _D_BE = 3, 63, 3, 10, 16
_C_GO, _C_TR, _C_EX, _C_BP = 0, 3, 6, 16
_R_GO, _R_BP, _R_TR, _R_EX = 8, 64, 8, 16

_NW = 32
_BPW = _B // _NW
_CH = 128
_GR = 256
_PW = 256
_NP = _F // _PW
_TO = _NP * _PW
_TW = _F - _TO
_PPT = -(-(_NP + 1) // _NW)

_TABLES = ((_C_GO, _D_GO), (_C_TR, _D_TR), (_C_EX, _D_EX), (_C_BP, _D_BP))

_mesh = plsc.VectorSubcoreMesh(core_axis_name="c", subcore_axis_name="s")
_params = pltpu.CompilerParams(use_tc_tiling_on_sc=True,
                               needs_layout_passes=False)


def _merge_body(go_hbm, tr_hbm, ex_hbm, bp_hbm, tail_hbm, mrg,
                go_pan, tr_pan, ex_pan, bp_pan, rowbuf, sem):
    wid = lax.axis_index("s") * 2 + lax.axis_index("c")
    iota = lax.iota(jnp.int32, 16)
    pans = ((go_pan, _C_GO, _D_GO), (tr_pan, _C_TR, _D_TR),
            (ex_pan, _C_EX, _D_EX), (bp_pan, _C_BP, _D_BP))

    def _transpose(nck):
        for pan, c0, d in pans:
            def _col(j, _, pan=pan, c0=c0):
                rowsj = jnp.full((16,), j, jnp.int32)
                colj = jnp.full((16,), c0 + j, jnp.int32)

                def _chunk(k, _):
                    lanes = k * 16 + iota
                    v = plsc.load_gather(pan, [rowsj, lanes])
                    plsc.store_scatter(rowbuf, [lanes, colj], v)
                    return 0

                lax.fori_loop(0, nck, _chunk, 0)
                return 0

            lax.fori_loop(0, d, _col, 0)

    def _stage(po, w):
        for src, pan in ((go_hbm, go_pan), (tr_hbm, tr_pan),
                         (ex_hbm, ex_pan), (bp_hbm, bp_pan)):
            pltpu.make_async_copy(src.at[:, pl.ds(po, w)],
                                  pan.at[:, pl.ds(0, w)] if w != _PW else pan,
                                  sem).start()
        for src, pan in ((go_hbm, go_pan), (tr_hbm, tr_pan),
                         (ex_hbm, ex_pan), (bp_hbm, bp_pan)):
            pltpu.make_async_copy(src.at[:, pl.ds(po, w)],
                                  pan.at[:, pl.ds(0, w)] if w != _PW else pan,
                                  sem).wait()

    def _panel(i, _):
        p = wid + i * _NW
        po = p * _PW

        @pl.when(p < _NP)
        def _do():
            _stage(po, _PW)
            _transpose(_PW // 16)
            pltpu.sync_copy(rowbuf, mrg.at[pl.ds(po, _PW)])

        @pl.when(p == _NP)
        def _tail():
            pltpu.sync_copy(tail_hbm, rowbuf.at[pl.ds(0, _TW)])
            pltpu.sync_copy(rowbuf.at[pl.ds(0, _TW)], mrg.at[pl.ds(_TO, _TW)])
        return 0

    lax.fori_loop(0, _PPT, _panel, 0)


def _gather_body(ids_hbm, be_hbm, mrg_hbm,
                 be_t, go_t, bp_t, tr_t, ex_t,
                 idx_v, be_row, rows_v, be_p, go_p, bp_p, tr_p, ex_p,
                 sem, wsem):
    wid = lax.axis_index("s") * 2 + lax.axis_index("c")
    base = wid * _BPW
    iota = lax.iota(jnp.int32, 16)

    pltpu.sync_copy(ids_hbm.at[pl.ds(base, _BPW)], idx_v)

    pltpu.sync_copy(be_hbm, be_row)
    vec = be_row[0, :]

    def _berow(d, _):
        val = jnp.sum(jnp.where(iota == d, vec, 0.0))
        bvec = jnp.full((16,), val, jnp.float32)

        def _fill(i, _):
            be_p[d, pl.ds(i * 16, 16)] = bvec
            return 0

        lax.fori_loop(0, _BPW // 16, _fill, 0)
        return 0

    lax.fori_loop(0, _D_BE, _berow, 0)
    writes = [pltpu.async_copy(be_p, be_t.at[:, pl.ds(base, _BPW)], wsem)]

    def _round(r, _):
        g1 = pltpu.async_copy(
            mrg_hbm.at[idx_v.at[pl.ds(r * _GR, _CH)]],
            rows_v.at[pl.ds(0, _CH)], sem)
        g2 = pltpu.async_copy(
            mrg_hbm.at[idx_v.at[pl.ds(r * _GR + _CH, _CH)]],
            rows_v.at[pl.ds(_CH, _CH)], sem)
        g1.wait()
        g2.wait()
        for dst, (c0, d) in zip((go_p, tr_p, ex_p, bp_p), _TABLES):
            def _col(j, _, dst=dst, c0=c0):
                colj = jnp.full((16,), c0 + j, jnp.int32)

                def _tile(i, _):
                    rows = i * 16 + iota
                    dst[j, pl.ds(r * _GR + i * 16, 16)] = plsc.load_gather(
                        rows_v, [rows, colj])
                    return 0

                lax.fori_loop(0, _GR // 16, _tile, 0)
                return 0

            lax.fori_loop(0, d, _col, 0)
        return 0

    lax.fori_loop(0, _BPW // _GR, _round, 0)

    for dst, out in ((go_p, go_t), (tr_p, tr_t), (ex_p, ex_t), (bp_p, bp_t)):
        writes.append(pltpu.async_copy(dst, out.at[:, pl.ds(base, _BPW)],
                                       wsem))
    for w in writes:
        w.wait()


_merge = pl.kernel(
    _merge_body,
    mesh=_mesh,
    compiler_params=_params,
    out_type=[jax.ShapeDtypeStruct((_F, _W), jnp.float32)],
    scratch_types=[
        pltpu.VMEM((_D_GO, _PW), jnp.float32),
        pltpu.VMEM((_D_TR, _PW), jnp.float32),
        pltpu.VMEM((_D_EX, _PW), jnp.float32),
        pltpu.VMEM((_D_BP, _PW), jnp.float32),
        pltpu.VMEM((_PW, _W), jnp.float32),
        pltpu.SemaphoreType.DMA,
    ],
)

_gather = pl.kernel(
    _gather_body,
    mesh=_mesh,
    compiler_params=_params,
    out_type=[
        jax.ShapeDtypeStruct((_D_BE, _B), jnp.float32),
        jax.ShapeDtypeStruct((_R_GO, _B), jnp.float32),
        jax.ShapeDtypeStruct((_R_BP, _B), jnp.float32),
        jax.ShapeDtypeStruct((_R_TR, _B), jnp.float32),
        jax.ShapeDtypeStruct((_R_EX, _B), jnp.float32),
    ],
    scratch_types=[
        pltpu.VMEM((_BPW,), jnp.int32),
        pltpu.VMEM((1, _D_BE), jnp.float32),
        pltpu.VMEM((_GR, _W), jnp.float32),
        pltpu.VMEM((_D_BE, _BPW), jnp.float32),
        pltpu.VMEM((_R_GO, _BPW), jnp.float32),
        pltpu.VMEM((_R_BP, _BPW), jnp.float32),
        pltpu.VMEM((_R_TR, _BPW), jnp.float32),
        pltpu.VMEM((_R_EX, _BPW), jnp.float32),
        pltpu.SemaphoreType.DMA,
        pltpu.SemaphoreType.DMA,
    ],
)


def kernel(frame_ids, betas_w, global_orient_w, body_pose_w, transl_w,
           expression_w):
    ids = frame_ids.astype(jnp.int32)
    tail = jnp.pad(
        jnp.concatenate([global_orient_w[-_TW:], transl_w[-_TW:],
                         expression_w[-_TW:], body_pose_w[-_TW:]], axis=1),
        ((0, 0), (0, _W - _C_BP - _D_BP)))
    (merged,) = _merge(global_orient_w.T, transl_w.T, expression_w.T,
                       body_pose_w.T, tail)
    be_t, go_t, bp_t, tr_t, ex_t = _gather(ids, betas_w, merged)
    return (be_t.T, go_t[:_D_GO].T, bp_t[:_D_BP].T, tr_t[:_D_TR].T,
            ex_t[:_D_EX].T)


# speedup vs baseline: 1.9656x; 1.0093x over previous
import jax
import jax.numpy as jnp
from jax import lax
from jax.experimental import pallas as pl
from jax.experimental.pallas import tpu as pltpu
from jax.experimental.pallas import tpu_sc as plsc

_F = 100000
_B = 16384
_W = 128
_D_GO, _D_BP, _D_TR, _D_EX, _D_BE = 3, 63, 3, 10, 16
_C_GO, _C_TR, _C_EX, _C_BP = 0, 3, 6, 16
_R_GO, _R_BP, _R_TR, _R_EX = 8, 64, 8, 16

_NW = 32
_BPW = _B // _NW
_CH = 128
_GR = 256
_PW = 256
_NP = _F // _PW
_TO = _NP * _PW
_TW = _F - _TO
_PPT = -(-(_NP + 1) // _NW)

_TABLES = ((_C_GO, _D_GO), (_C_TR, _D_TR), (_C_EX, _D_EX), (_C_BP, _D_BP))

_mesh = plsc.VectorSubcoreMesh(core_axis_name="c", subcore_axis_name="s")
_params = pltpu.CompilerParams(use_tc_tiling_on_sc=True,
                               needs_layout_passes=False)


def _merge_body(go_hbm, tr_hbm, ex_hbm, bp_hbm, tail_hbm, mrg,
                go_pan, tr_pan, ex_pan, bp_pan, rowbuf, sem):
    wid = lax.axis_index("s") * 2 + lax.axis_index("c")
    iota = lax.iota(jnp.int32, 16)
    pans = ((go_pan, _C_GO, _D_GO), (tr_pan, _C_TR, _D_TR),
            (ex_pan, _C_EX, _D_EX), (bp_pan, _C_BP, _D_BP))

    def _transpose(nck):
        for pan, c0, d in pans:
            def _col(j, _, pan=pan, c0=c0):
                rowsj = jnp.full((16,), j, jnp.int32)
                colj = jnp.full((16,), c0 + j, jnp.int32)
                for k in range(nck):
                    lanes = k * 16 + iota
                    v = plsc.load_gather(pan, [rowsj, lanes])
                    plsc.store_scatter(rowbuf, [lanes, colj], v)
                return 0

            lax.fori_loop(0, d, _col, 0)

    def _stage(po, w):
        for src, pan in ((go_hbm, go_pan), (tr_hbm, tr_pan),
                         (ex_hbm, ex_pan), (bp_hbm, bp_pan)):
            pltpu.make_async_copy(src.at[:, pl.ds(po, w)],
                                  pan.at[:, pl.ds(0, w)] if w != _PW else pan,
                                  sem).start()
        for src, pan in ((go_hbm, go_pan), (tr_hbm, tr_pan),
                         (ex_hbm, ex_pan), (bp_hbm, bp_pan)):
            pltpu.make_async_copy(src.at[:, pl.ds(po, w)],
                                  pan.at[:, pl.ds(0, w)] if w != _PW else pan,
                                  sem).wait()

    def _panel(i, _):
        p = wid + i * _NW
        po = p * _PW

        @pl.when(p < _NP)
        def _do():
            _stage(po, _PW)
            _transpose(_PW // 16)
            pltpu.sync_copy(rowbuf, mrg.at[pl.ds(po, _PW)])

        @pl.when(p == _NP)
        def _tail():
            pltpu.sync_copy(tail_hbm, rowbuf.at[pl.ds(0, _TW)])
            pltpu.sync_copy(rowbuf.at[pl.ds(0, _TW)], mrg.at[pl.ds(_TO, _TW)])
        return 0

    lax.fori_loop(0, _PPT, _panel, 0)


def _gather_body(ids_hbm, be_hbm, mrg_hbm,
                 be_t, go_t, bp_t, tr_t, ex_t,
                 idx_v, be_row, rows_v, be_p, go_p, bp_p, tr_p, ex_p,
                 sem, wsem):
    wid = lax.axis_index("s") * 2 + lax.axis_index("c")
    base = wid * _BPW
    iota = lax.iota(jnp.int32, 16)

    pltpu.sync_copy(ids_hbm.at[pl.ds(base, _BPW)], idx_v)

    pltpu.sync_copy(be_hbm, be_row)
    vec = be_row[0, :]

    def _berow(d, _):
        val = jnp.sum(jnp.where(iota == d, vec, 0.0))
        bvec = jnp.full((16,), val, jnp.float32)

        def _fill(i, _):
            be_p[d, pl.ds(i * 16, 16)] = bvec
            return 0

        lax.fori_loop(0, _BPW // 16, _fill, 0)
        return 0

    lax.fori_loop(0, _D_BE, _berow, 0)
    writes = [pltpu.async_copy(be_p, be_t.at[:, pl.ds(base, _BPW)], wsem)]

    def _round(r, _):
        g1 = pltpu.async_copy(
            mrg_hbm.at[idx_v.at[pl.ds(r * _GR, _CH)]],
            rows_v.at[pl.ds(0, _CH)], sem)
        g2 = pltpu.async_copy(
            mrg_hbm.at[idx_v.at[pl.ds(r * _GR + _CH, _CH)]],
            rows_v.at[pl.ds(_CH, _CH)], sem)
        g1.wait()
        g2.wait()
        for dst, (c0, d) in zip((go_p, tr_p, ex_p, bp_p), _TABLES):
            def _col(j, _, dst=dst, c0=c0):
                colj = jnp.full((16,), c0 + j, jnp.int32)
                for i in range(_GR // 16):
                    rows = i * 16 + iota
                    dst[j, pl.ds(r * _GR + i * 16, 16)] = plsc.load_gather(
                        rows_v, [rows, colj])
                return 0

            lax.fori_loop(0, d, _col, 0)
        return 0

    lax.fori_loop(0, _BPW // _GR, _round, 0)

    for dst, out in ((go_p, go_t), (tr_p, tr_t), (ex_p, ex_t), (bp_p, bp_t)):
        writes.append(pltpu.async_copy(dst, out.at[:, pl.ds(base, _BPW)],
                                       wsem))
    for w in writes:
        w.wait()


_merge = pl.kernel(
    _merge_body,
    mesh=_mesh,
    compiler_params=_params,
    out_type=[jax.ShapeDtypeStruct((_F, _W), jnp.float32)],
    scratch_types=[
        pltpu.VMEM((_D_GO, _PW), jnp.float32),
        pltpu.VMEM((_D_TR, _PW), jnp.float32),
        pltpu.VMEM((_D_EX, _PW), jnp.float32),
        pltpu.VMEM((_D_BP, _PW), jnp.float32),
        pltpu.VMEM((_PW, _W), jnp.float32),
        pltpu.SemaphoreType.DMA,
    ],
)

_gather = pl.kernel(
    _gather_body,
    mesh=_mesh,
    compiler_params=_params,
    out_type=[
        jax.ShapeDtypeStruct((_D_BE, _B), jnp.float32),
        jax.ShapeDtypeStruct((_R_GO, _B), jnp.float32),
        jax.ShapeDtypeStruct((_R_BP, _B), jnp.float32),
        jax.ShapeDtypeStruct((_R_TR, _B), jnp.float32),
        jax.ShapeDtypeStruct((_R_EX, _B), jnp.float32),
    ],
    scratch_types=[
        pltpu.VMEM((_BPW,), jnp.int32),
        pltpu.VMEM((1, _D_BE), jnp.float32),
        pltpu.VMEM((_GR, _W), jnp.float32),
        pltpu.VMEM((_D_BE, _BPW), jnp.float32),
        pltpu.VMEM((_R_GO, _BPW), jnp.float32),
        pltpu.VMEM((_R_BP, _BPW), jnp.float32),
        pltpu.VMEM((_R_TR, _BPW), jnp.float32),
        pltpu.VMEM((_R_EX, _BPW), jnp.float32),
        pltpu.SemaphoreType.DMA,
        pltpu.SemaphoreType.DMA,
    ],
)


def kernel(frame_ids, betas_w, global_orient_w, body_pose_w, transl_w,
           expression_w):
    ids = frame_ids.astype(jnp.int32)
    tail = jnp.pad(
        jnp.concatenate([global_orient_w[-_TW:], transl_w[-_TW:],
                         expression_w[-_TW:], body_pose_w[-_TW:]], axis=1),
        ((0, 0), (0, _W - _C_BP - _D_BP)))
    (merged,) = _merge(global_orient_w.T, transl_w.T, expression_w.T,
                       body_pose_w.T, tail)
    be_t, go_t, bp_t, tr_t, ex_t = _gather(ids, betas_w, merged)
    return (be_t.T, go_t[:_D_GO].T, bp_t[:_D_BP].T, tr_t[:_D_TR].T,
            ex_t[:_D_EX].T)


# speedup vs baseline: 2.2197x; 1.1293x over previous
import jax
import jax.numpy as jnp
from jax import lax
from jax.experimental import pallas as pl
from jax.experimental.pallas import tpu as pltpu
from jax.experimental.pallas import tpu_sc as plsc

_F = 100000
_B = 16384
_W = 128
_D_GO, _D_BP, _D_TR, _D_EX, _D_BE = 3, 63, 3, 10, 16
_C_GO, _C_TR, _C_EX, _C_BP = 0, 3, 6, 16
_R_GO, _R_BP, _R_TR, _R_EX = 8, 64, 8, 16

_NW = 32
_BPW = _B // _NW
_CH = 128
_NCH = _BPW // _CH
_NCK = _BPW // 16

_mesh = plsc.VectorSubcoreMesh(core_axis_name="c", subcore_axis_name="s")


def _body(ids_hbm, be_hbm, tbl_hbm,
          be_t, go_t, bp_t, tr_t, ex_t,
          idx_v, be_row, rows_v, be_p, go_p, bp_p, tr_p, ex_p, sem, wsem):
    wid = lax.axis_index("s") * 2 + lax.axis_index("c")
    base = wid * _BPW

    pltpu.sync_copy(ids_hbm.at[pl.ds(wid * _NCH, _NCH)], idx_v)
    gathers = [pltpu.async_copy(tbl_hbm.at[idx_v.at[j]],
                                rows_v.at[pl.ds(j * _CH, _CH)], sem)
               for j in range(_NCH)]

    pltpu.sync_copy(be_hbm, be_row)
    vec = be_row[0, :]
    lanes = lax.iota(jnp.int32, 16)
    for d in range(_D_BE):
        val = jnp.sum(jnp.where(lanes == d, vec, 0.0))
        bvec = jnp.full((16,), val, jnp.float32)

        def _fill(i, _, d=d, bvec=bvec):
            be_p[d, pl.ds(i * 16, 16)] = bvec
            return 0

        lax.fori_loop(0, _NCK, _fill, 0)
    writes = [pltpu.async_copy(be_p, be_t.at[:, pl.ds(base, _BPW)], wsem)]

    for g in gathers:
        g.wait()

    iota = lax.iota(jnp.int32, 16)
    for dst, out, c0, d in ((go_p, go_t, _C_GO, _D_GO),
                            (tr_p, tr_t, _C_TR, _D_TR),
                            (ex_p, ex_t, _C_EX, _D_EX),
                            (bp_p, bp_t, _C_BP, _D_BP)):
        def _tile(i, _, dst=dst, c0=c0, d=d):
            rows = i * 16 + iota
            for j in range(d):
                cols = jnp.full((16,), c0 + j, jnp.int32)
                dst[j, pl.ds(i * 16, 16)] = plsc.load_gather(
                    rows_v, [rows, cols])
            return 0

        lax.fori_loop(0, _NCK, _tile, 0)
        writes.append(pltpu.async_copy(dst, out.at[:, pl.ds(base, _BPW)],
                                       wsem))
    for w in writes:
        w.wait()


def _build(interpret=False):
    return pl.kernel(
        _body,
        mesh=_mesh,
        compiler_params=pltpu.CompilerParams(use_tc_tiling_on_sc=False,
                                             needs_layout_passes=False),
        out_type=[
            jax.ShapeDtypeStruct((_D_BE, _B), jnp.float32),
            jax.ShapeDtypeStruct((_R_GO, _B), jnp.float32),
            jax.ShapeDtypeStruct((_R_BP, _B), jnp.float32),
            jax.ShapeDtypeStruct((_R_TR, _B), jnp.float32),
            jax.ShapeDtypeStruct((_R_EX, _B), jnp.float32),
        ],
        scratch_types=[
            pltpu.VMEM((_NCH, _CH), jnp.int32),
            pltpu.VMEM((1, _D_BE), jnp.float32),
            pltpu.VMEM((_BPW, _W), jnp.float32),
            pltpu.VMEM((_D_BE, _BPW), jnp.float32),
            pltpu.VMEM((_R_GO, _BPW), jnp.float32),
            pltpu.VMEM((_R_BP, _BPW), jnp.float32),
            pltpu.VMEM((_R_TR, _BPW), jnp.float32),
            pltpu.VMEM((_R_EX, _BPW), jnp.float32),
            pltpu.SemaphoreType.DMA,
            pltpu.SemaphoreType.DMA,
        ],
        interpret=interpret,
    )


_gather_all = _build()


def kernel(frame_ids, betas_w, global_orient_w, body_pose_w, transl_w,
           expression_w):
    ids2d = frame_ids.astype(jnp.int32).reshape(_B // _CH, _CH)
    merged = jnp.pad(
        jnp.concatenate([global_orient_w, transl_w, expression_w,
                         body_pose_w], axis=1),
        ((0, 0), (0, _W - _C_BP - _D_BP)))
    be_t, go_t, bp_t, tr_t, ex_t = _gather_all(ids2d, betas_w, merged)
    return (be_t.T, go_t[:_D_GO].T, bp_t[:_D_BP].T, tr_t[:_D_TR].T,
            ex_t[:_D_EX].T)
